# Initial kernel scaffold; baseline (speedup 1.0000x reference)
import jax
import jax.numpy as jnp
from jax.experimental import pallas as pl


def kernel(edge_label_index, adj_row, adj_col, adj_value, emb, gc_w0, gc_b0, bi_w0, bi_b0, gc_w1, gc_b1, bi_w1, bi_b1, gc_w2, gc_b2, bi_w2, bi_b2):
    raise NotImplementedError("write your pallas kernel here")


# speedup vs baseline: 4.5577x; 4.5577x over previous
import functools

import jax
import jax.numpy as jnp
from jax import lax
from jax.experimental import pallas as pl
from jax.experimental.pallas import tpu as pltpu
from jax.experimental.pallas import tpu_sc as plsc

N_TOTAL = 50000
EMB = 64
LAYERS = 3
E_EDGES = 800000
B_PAIRS = 4096

NC = 2
NS = 16
L = 16

R_HALF = 25088
N_PAD = 2 * R_HALF
ROWS_PER_TILE = R_HALF // NS
K_EDGE = 128
NB = E_EDGES // K_EDGE
PAIRS_PER_TILE = B_PAIRS // (NC * NS)

_MESH = plsc.VectorSubcoreMesh(core_axis_name="c", subcore_axis_name="s")


@functools.partial(
    pl.kernel,
    mesh=_MESH,
    out_type=jax.ShapeDtypeStruct((N_PAD, EMB), jnp.float32),
    scratch_types=[
        pltpu.VMEM((L,), jnp.int32),
        pltpu.VMEM((K_EDGE,), jnp.int32),
        pltpu.VMEM((K_EDGE,), jnp.int32),
        pltpu.VMEM((K_EDGE,), jnp.float32),
        pltpu.VMEM((K_EDGE,), jnp.int32),
        pltpu.VMEM((K_EDGE, EMB), jnp.float32),
        pltpu.VMEM((K_EDGE, EMB), jnp.float32),
        pltpu.SemaphoreType.DMA,
        pltpu.VMEM_SHARED((R_HALF + 8, EMB), jnp.float32),
    ],
    compiler_params=pltpu.CompilerParams(use_tc_tiling_on_sc=False, needs_layout_passes=False),
)
def _spmm(params_hbm, col_hbm, row_hbm, val_hbm, ego_hbm, msg_hbm,
          params_v, col_v, row_v, val_v, idx_v, rows_v, zero_v, sem, acc_sh):
    cid = lax.axis_index("c")
    sid = lax.axis_index("s")
    base_row = cid * R_HALF

    z = jnp.zeros((L,), jnp.float32)
    for j in range(K_EDGE):
        for c in range(EMB // L):
            zero_v[j, pl.ds(c * L, L)] = z
    my0 = sid * ROWS_PER_TILE
    nfull = ROWS_PER_TILE // K_EDGE
    rem = ROWS_PER_TILE - nfull * K_EDGE
    for t in range(nfull):
        pltpu.sync_copy(zero_v, acc_sh.at[pl.ds(my0 + t * K_EDGE, K_EDGE)])
    pltpu.sync_copy(zero_v.at[pl.ds(0, rem)], acc_sh.at[pl.ds(my0 + nfull * K_EDGE, rem)])
    pltpu.sync_copy(zero_v.at[pl.ds(0, 8)], acc_sh.at[pl.ds(R_HALF, 8)])
    plsc.subcore_barrier()

    pltpu.sync_copy(params_hbm, params_v)
    e_mid = params_v[pl.ds(0, L)][0]
    b_lo = e_mid // K_EDGE
    b_hi = (e_mid + K_EDGE - 1) // K_EDGE
    b_start = jnp.where(cid == 0, 0, b_lo)
    b_end = jnp.where(cid == 0, b_hi, NB)
    first = b_start + sid
    n_it = jnp.maximum(0, (b_end - first + NS - 1) // NS)

    def block_body(i, carry):
        b = first + i * NS
        off = b * K_EDGE
        pltpu.sync_copy(col_hbm.at[pl.ds(off, K_EDGE)], col_v)
        pltpu.sync_copy(row_hbm.at[pl.ds(off, K_EDGE)], row_v)
        pltpu.sync_copy(val_hbm.at[pl.ds(off, K_EDGE)], val_v)
        pltpu.async_copy(ego_hbm.at[col_v], rows_v, sem).wait()
        for c in range(K_EDGE // L):
            r = row_v[pl.ds(c * L, L)]
            lr = r - base_row
            ok = (lr >= 0) & (lr < R_HALF)
            idx_v[pl.ds(c * L, L)] = jnp.where(ok, lr, R_HALF)
        for g in range(K_EDGE // L):
            v16 = val_v[pl.ds(g * L, L)]
            for k in range(L):
                j = g * L + k
                vb = jnp.broadcast_to(v16[k], (L,))
                for c in range(EMB // L):
                    x = rows_v[j, pl.ds(c * L, L)]
                    rows_v[j, pl.ds(c * L, L)] = x * vb
        pltpu.sync_copy(rows_v, acc_sh.at[idx_v], add=True)
        return carry

    lax.fori_loop(0, n_it, block_body, 0)
    plsc.subcore_barrier()

    pltpu.sync_copy(acc_sh.at[pl.ds(my0, ROWS_PER_TILE)],
                    msg_hbm.at[pl.ds(base_row + my0, ROWS_PER_TILE)])


def _dense_body(msg_ref, ego_ref, gw_ref, gb_ref, bw_ref, bb_ref,
                ego_out_ref, norm_ref):
    msg = msg_ref[...]
    ego = ego_ref[...]
    aggr = lax.dot_general(msg, gw_ref[...], (((1,), (1,)), ((), ())),
                           preferred_element_type=jnp.float32) + gb_ref[...]
    bi = lax.dot_general(ego * msg, bw_ref[...], (((1,), (1,)), ((), ())),
                         preferred_element_type=jnp.float32) + bb_ref[...]
    h = aggr + bi
    h = jnp.where(h >= 0, h, 0.2 * h)
    ego_out_ref[...] = h
    n = jnp.sqrt(jnp.sum(h * h, axis=1, keepdims=True))
    norm_ref[...] = h / jnp.maximum(n, 1e-12)


TC_BLK = 512

_dense = pl.pallas_call(
    _dense_body,
    grid=(N_PAD // TC_BLK,),
    in_specs=[
        pl.BlockSpec((TC_BLK, EMB), lambda i: (i, 0)),
        pl.BlockSpec((TC_BLK, EMB), lambda i: (i, 0)),
        pl.BlockSpec((EMB, EMB), lambda i: (0, 0)),
        pl.BlockSpec((1, EMB), lambda i: (0, 0)),
        pl.BlockSpec((EMB, EMB), lambda i: (0, 0)),
        pl.BlockSpec((1, EMB), lambda i: (0, 0)),
    ],
    out_specs=[
        pl.BlockSpec((TC_BLK, EMB), lambda i: (i, 0)),
        pl.BlockSpec((TC_BLK, EMB), lambda i: (i, 0)),
    ],
    out_shape=[
        jax.ShapeDtypeStruct((N_PAD, EMB), jnp.float32),
        jax.ShapeDtypeStruct((N_PAD, EMB), jnp.float32),
    ],
)


@functools.partial(
    pl.kernel,
    mesh=_MESH,
    out_type=jax.ShapeDtypeStruct((B_PAIRS,), jnp.float32),
    scratch_types=[
        pltpu.VMEM((PAIRS_PER_TILE,), jnp.int32),
        pltpu.VMEM((PAIRS_PER_TILE,), jnp.int32),
        pltpu.VMEM((PAIRS_PER_TILE,), jnp.float32),
        pltpu.VMEM((PAIRS_PER_TILE, EMB), jnp.float32),
        pltpu.VMEM((PAIRS_PER_TILE, EMB), jnp.float32),
        pltpu.VMEM((PAIRS_PER_TILE, EMB), jnp.float32),
        pltpu.VMEM((PAIRS_PER_TILE, EMB), jnp.float32),
        pltpu.VMEM((PAIRS_PER_TILE, EMB), jnp.float32),
        pltpu.VMEM((PAIRS_PER_TILE, EMB), jnp.float32),
        pltpu.VMEM((PAIRS_PER_TILE, EMB), jnp.float32),
        pltpu.VMEM((PAIRS_PER_TILE, EMB), jnp.float32),
        pltpu.SemaphoreType.DMA,
    ],
    compiler_params=pltpu.CompilerParams(use_tc_tiling_on_sc=False, needs_layout_passes=False),
)
def _pair_dot(si_hbm, di_hbm, t0, t1, t2, t3, out_hbm,
              is_v, id_v, out_v, rs0, rs1, rs2, rs3, rd0, rd1, rd2, rd3, sem):
    cid = lax.axis_index("c")
    sid = lax.axis_index("s")
    wid = sid * NC + cid
    base = wid * PAIRS_PER_TILE
    pltpu.sync_copy(si_hbm.at[pl.ds(base, PAIRS_PER_TILE)], is_v)
    pltpu.sync_copy(di_hbm.at[pl.ds(base, PAIRS_PER_TILE)], id_v)
    for tbl, rs, rd in ((t0, rs0, rd0), (t1, rs1, rd1), (t2, rs2, rd2), (t3, rs3, rd3)):
        pltpu.async_copy(tbl.at[is_v], rs, sem).wait()
        pltpu.async_copy(tbl.at[id_v], rd, sem).wait()

    lane = lax.iota(jnp.int32, L)
    for g in range(PAIRS_PER_TILE // L):
        pair = lane + g * L
        acc = jnp.zeros((L,), jnp.float32)
        for rs, rd in ((rs0, rd0), (rs1, rd1), (rs2, rd2), (rs3, rd3)):
            for c in range(EMB):
                ccol = jnp.full((L,), c, jnp.int32)
                a = plsc.load_gather(rs, [pair, ccol])
                b = plsc.load_gather(rd, [pair, ccol])
                acc = acc + a * b
        out_v[pl.ds(g * L, L)] = acc
    pltpu.sync_copy(out_v, out_hbm.at[pl.ds(base, PAIRS_PER_TILE)])


def kernel(edge_label_index, adj_row, adj_col, adj_value, emb,
           gc_w0, gc_b0, bi_w0, bi_b0,
           gc_w1, gc_b1, bi_w1, bi_b1,
           gc_w2, gc_b2, bi_w2, bi_b2):
    e_mid = jnp.searchsorted(adj_row, jnp.int32(R_HALF), side="left").astype(jnp.int32)
    params = jnp.zeros((L,), jnp.int32).at[0].set(e_mid)
    ego = jnp.zeros((N_PAD, EMB), jnp.float32).at[:N_TOTAL].set(emb)
    gc = [(gc_w0, gc_b0), (gc_w1, gc_b1), (gc_w2, gc_b2)]
    bi = [(bi_w0, bi_b0), (bi_w1, bi_b1), (bi_w2, bi_b2)]
    tables = [ego]
    for i in range(LAYERS):
        msg = _spmm(params, adj_col, adj_row, adj_value, ego)
        ego, norm = _dense(msg, ego, gc[i][0], gc[i][1].reshape(1, EMB),
                           bi[i][0], bi[i][1].reshape(1, EMB))
        tables.append(norm)
    return _pair_dot(edge_label_index[0], edge_label_index[1],
                     tables[0], tables[1], tables[2], tables[3])
